# baseline (speedup 1.0000x reference)
import jax
import jax.numpy as jnp
from jax import lax
from jax.experimental import pallas as pl
from jax.experimental.pallas import tpu as pltpu
from jax.experimental.pallas import tpu_sc as plsc

NC = 2
NS = 16
CHUNK = 128
CH_ALIGN = 160
MB = 256


def _zero_accum_slice(rows0, accum, base_r, n_row_blk, nz, d):
  def zr(i, _):
    rows0[i // (d // 16), pl.ds((i % (d // 16)) * 16, 16)] = jnp.zeros(
        (16,), jnp.float32)
    return 0
  lax.fori_loop(0, nz, zr, 0)

  def zb(k, _):
    pltpu.sync_copy(rows0, accum.at[pl.ds(base_r + k * CHUNK, CHUNK)])
    return 0
  lax.fori_loop(0, n_row_blk, zb, 0)


def _writeback(rows0, accum, out_ref, base_r, n_row_blk):
  def wb(k, _):
    r0 = base_r + k * CHUNK
    pltpu.sync_copy(accum.at[pl.ds(r0, CHUNK)], rows0)
    pltpu.sync_copy(rows0, out_ref.at[pl.ds(r0, CHUNK)])
    return 0
  lax.fori_loop(0, n_row_blk, wb, 0)


def _sc_edge_scatter_cols(table2, src2d, dst2d, npad, dh, ch_sub, nslot):
  rows_per_sub = npad // NS
  n_row_blk = rows_per_sub // CHUNK
  nz = CHUNK * (dh // 16)
  assert ch_sub % nslot == 0
  ngrp = ch_sub // nslot

  def body(table_hbm, src_hbm, dst_hbm, out_hbm, idx_s, idx_d, *rest):
    rows = list(rest[:nslot])
    accum = rest[nslot]
    gsem = list(rest[nslot + 1:2 * nslot + 1])
    cid = lax.axis_index("c")
    sid = lax.axis_index("s")
    base_r = sid * rows_per_sub
    base_c = sid * ch_sub

    _zero_accum_slice(rows[0], accum, base_r, n_row_blk, nz, dh)

    pltpu.sync_copy(src_hbm.at[pl.ds(base_c, ch_sub)], idx_s)
    pltpu.sync_copy(dst_hbm.at[pl.ds(base_c, ch_sub)], idx_d)
    plsc.subcore_barrier()

    def fire_g(j, b):
      pltpu.async_copy(table_hbm.at[cid].at[idx_s.at[j]], rows[b], gsem[b])

    def wait_g(j, b):
      pltpu.make_async_copy(table_hbm.at[cid].at[idx_s.at[j]], rows[b],
                            gsem[b]).wait()

    for b in range(nslot):
      fire_g(b, b)

    def grp(g, _):
      j0 = g * nslot
      for b in range(nslot):
        wait_g(j0 + b, b)
        pltpu.sync_copy(rows[b], accum.at[idx_d.at[j0 + b]], add=True)
        jn = jnp.minimum(j0 + nslot + b, ch_sub - 1)
        fire_g(jn, b)
      return 0
    lax.fori_loop(0, ngrp, grp, 0)
    for b in range(nslot):
      wait_g(ch_sub - 1, b)
    plsc.subcore_barrier()

    _writeback(rows[0], accum, out_hbm.at[cid], base_r, n_row_blk)

  return pl.kernel(
      body,
      out_type=jax.ShapeDtypeStruct((NC, npad, dh), jnp.float32),
      mesh=plsc.VectorSubcoreMesh(core_axis_name="c", subcore_axis_name="s"),
      compiler_params=pltpu.CompilerParams(use_tc_tiling_on_sc=False),
      scratch_types=(
          [pltpu.VMEM((ch_sub, CHUNK), jnp.int32),
           pltpu.VMEM((ch_sub, CHUNK), jnp.int32)]
          + [pltpu.VMEM((CHUNK, dh), jnp.float32) for _ in range(nslot)]
          + [pltpu.VMEM_SHARED((npad, dh), jnp.float32)]
          + [pltpu.SemaphoreType.DMA for _ in range(nslot)]
      ),
  )(table2, src2d, dst2d)


def _sc_edge_scatter(table, src2d, dst2d, npad, d, ch_sub, nslot):
  rows_per_sub = npad // NS
  n_row_blk = rows_per_sub // CHUNK
  nz = CHUNK * (d // 16)
  chw = ch_sub // NC
  assert chw % nslot == 0
  ngrp = chw // nslot

  def body(table_hbm, src_hbm, dst_hbm, out_hbm, idx_s, idx_d, *rest):
    rows = list(rest[:nslot])
    accum = rest[nslot]
    gsem = list(rest[nslot + 1:2 * nslot + 1])
    cid = lax.axis_index("c")
    sid = lax.axis_index("s")
    base_r = sid * rows_per_sub
    base_c = sid * ch_sub + cid * chw

    _zero_accum_slice(rows[0], accum, base_r, n_row_blk, nz, d)

    pltpu.sync_copy(src_hbm.at[pl.ds(base_c, chw)], idx_s)
    pltpu.sync_copy(dst_hbm.at[pl.ds(base_c, chw)], idx_d)
    plsc.subcore_barrier()

    def fire_g(j, b):
      pltpu.async_copy(table_hbm.at[idx_s.at[j]], rows[b], gsem[b])

    def wait_g(j, b):
      pltpu.make_async_copy(table_hbm.at[idx_s.at[j]], rows[b],
                            gsem[b]).wait()

    for b in range(nslot):
      fire_g(b, b)

    def grp(g, _):
      j0 = g * nslot
      for b in range(nslot):
        wait_g(j0 + b, b)
        pltpu.sync_copy(rows[b], accum.at[idx_d.at[j0 + b]], add=True)
        jn = jnp.minimum(j0 + nslot + b, chw - 1)
        fire_g(jn, b)
      return 0
    lax.fori_loop(0, ngrp, grp, 0)
    for b in range(nslot):
      wait_g(chw - 1, b)
    plsc.subcore_barrier()

    _writeback(rows[0], accum, out_hbm.at[cid], base_r, n_row_blk)

  return pl.kernel(
      body,
      out_type=jax.ShapeDtypeStruct((NC, npad, d), jnp.float32),
      mesh=plsc.VectorSubcoreMesh(core_axis_name="c", subcore_axis_name="s"),
      compiler_params=pltpu.CompilerParams(use_tc_tiling_on_sc=False),
      scratch_types=(
          [pltpu.VMEM((chw, CHUNK), jnp.int32),
           pltpu.VMEM((chw, CHUNK), jnp.int32)]
          + [pltpu.VMEM((CHUNK, d), jnp.float32) for _ in range(nslot)]
          + [pltpu.VMEM_SHARED((npad, d), jnp.float32)]
          + [pltpu.SemaphoreType.DMA for _ in range(nslot)]
      ),
  )(table, src2d, dst2d)


def _sc_degree(dst2d, npad, ch_sub):
  d = 16
  rows_per_sub = npad // NS
  n_row_blk = rows_per_sub // CHUNK
  chw = ch_sub // NC

  def body(dst_hbm, out_hbm, idx_d, rows, accum):
    cid = lax.axis_index("c")
    sid = lax.axis_index("s")
    base_r = sid * rows_per_sub
    base_c = sid * ch_sub + cid * chw

    _zero_accum_slice(rows, accum, base_r, n_row_blk, CHUNK, d)

    def on(i, _):
      rows[i, pl.ds(0, 16)] = jnp.ones((16,), jnp.float32)
      return 0
    lax.fori_loop(0, CHUNK, on, 0)

    pltpu.sync_copy(dst_hbm.at[pl.ds(base_c, chw)], idx_d)
    plsc.subcore_barrier()

    def step(j, _):
      pltpu.sync_copy(rows, accum.at[idx_d.at[j]], add=True)
      return 0
    lax.fori_loop(0, chw, step, 0)
    plsc.subcore_barrier()

    _writeback(rows, accum, out_hbm.at[cid], base_r, n_row_blk)

  return pl.kernel(
      body,
      out_type=jax.ShapeDtypeStruct((NC, npad, d), jnp.float32),
      mesh=plsc.VectorSubcoreMesh(core_axis_name="c", subcore_axis_name="s"),
      compiler_params=pltpu.CompilerParams(use_tc_tiling_on_sc=False),
      scratch_types=[
          pltpu.VMEM((chw, CHUNK), jnp.int32),
          pltpu.VMEM((CHUNK, d), jnp.float32),
          pltpu.VMEM_SHARED((npad, d), jnp.float32),
      ],
  )(dst2d)


def _dinv_of(dp_ref):
  return lax.rsqrt(dp_ref[0, :, 0:1] + dp_ref[1, :, 0:1] + 1.0)


def _tc_matmul1(x_pad, w1, npad, f, h):
  def body(xr, w1r, m1):
    m1[...] = jnp.dot(xr[...], w1r[...], preferred_element_type=jnp.float32)
  return pl.pallas_call(
      body,
      grid=(npad // MB,),
      in_specs=[
          pl.BlockSpec((MB, f), lambda i: (i, 0)),
          pl.BlockSpec((f, h), lambda i: (0, 0)),
      ],
      out_specs=pl.BlockSpec((MB, h), lambda i: (i, 0)),
      out_shape=jax.ShapeDtypeStruct((npad, h), jnp.float32),
  )(x_pad, w1)


def _tc_scale1(deg_parts, m1, npad, h):
  hh = h // 2

  def body(dp, mr, g1o):
    dinv = _dinv_of(dp)
    g1 = dinv * mr[...]
    g1o[0] = g1[:, :hh]
    g1o[1] = g1[:, hh:]
  return pl.pallas_call(
      body,
      grid=(npad // MB,),
      in_specs=[
          pl.BlockSpec((NC, MB, 16), lambda i: (0, i, 0)),
          pl.BlockSpec((MB, h), lambda i: (i, 0)),
      ],
      out_specs=pl.BlockSpec((NC, MB, hh), lambda i: (0, i, 0)),
      out_shape=jax.ShapeDtypeStruct((NC, npad, hh), jnp.float32),
  )(deg_parts, m1)


def _tc_layer2(deg_parts, s1, g1, b1, w2, npad, h, c):
  hh = h // 2

  def body(dp, s1r, g1r, b1r, w2r, g2):
    dinv = _dinv_of(dp)
    m = jnp.concatenate([s1r[0] + g1r[0], s1r[1] + g1r[1]], axis=1)
    h1 = jnp.maximum(dinv * m + b1r[...], 0.0)
    g2[...] = dinv * jnp.dot(h1, w2r[...], preferred_element_type=jnp.float32)
  return pl.pallas_call(
      body,
      grid=(npad // MB,),
      in_specs=[
          pl.BlockSpec((NC, MB, 16), lambda i: (0, i, 0)),
          pl.BlockSpec((NC, MB, hh), lambda i: (0, i, 0)),
          pl.BlockSpec((NC, MB, hh), lambda i: (0, i, 0)),
          pl.BlockSpec((1, h), lambda i: (0, 0)),
          pl.BlockSpec((h, c), lambda i: (0, 0)),
      ],
      out_specs=pl.BlockSpec((MB, c), lambda i: (i, 0)),
      out_shape=jax.ShapeDtypeStruct((npad, c), jnp.float32),
  )(deg_parts, s1, g1, b1, w2)


def _tc_final(deg_parts, s2, g2, b2, npad, c):
  def body(dp, s2r, g2r, b2r, o):
    dinv = _dinv_of(dp)
    o[...] = dinv * (s2r[0] + s2r[1] + g2r[...]) + b2r[...]
  return pl.pallas_call(
      body,
      grid=(npad // MB,),
      in_specs=[
          pl.BlockSpec((NC, MB, 16), lambda i: (0, i, 0)),
          pl.BlockSpec((NC, MB, c), lambda i: (0, i, 0)),
          pl.BlockSpec((MB, c), lambda i: (i, 0)),
          pl.BlockSpec((1, c), lambda i: (0, 0)),
      ],
      out_specs=pl.BlockSpec((MB, c), lambda i: (i, 0)),
      out_shape=jax.ShapeDtypeStruct((npad, c), jnp.float32),
  )(deg_parts, s2, g2, b2)


def kernel(x, edge_index, W1, b1, W2, b2):
  n, f = x.shape
  h = W1.shape[1]
  c = W2.shape[1]
  e = edge_index.shape[1]

  npad = -(-(n + 1) // (NS * CHUNK)) * (NS * CHUNK)
  ch_min = -(-e // (NS * CHUNK))
  ch_sub = -(-ch_min // CH_ALIGN) * CH_ALIGN
  erows = ch_sub * NS
  epad = erows * CHUNK

  src = edge_index[0]
  dst = edge_index[1]
  pad_idx = jnp.full((epad - e,), n, dtype=jnp.int32)
  src2d = jnp.concatenate([src, pad_idx]).reshape(erows, CHUNK)
  dst2d = jnp.concatenate([dst, pad_idx]).reshape(erows, CHUNK)
  x_pad = jnp.pad(x, ((0, npad - n), (0, 0)))

  deg_parts = _sc_degree(dst2d, npad, ch_sub)
  m1 = _tc_matmul1(x_pad, W1, npad, f, h)
  g1 = _tc_scale1(deg_parts, m1, npad, h)
  s1 = _sc_edge_scatter_cols(g1, src2d, dst2d, npad, h // 2, ch_sub, 5)
  g2 = _tc_layer2(deg_parts, s1, g1, b1.reshape(1, h), W2, npad, h, c)
  s2 = _sc_edge_scatter(g2, src2d, dst2d, npad, c, ch_sub, 16)
  out = _tc_final(deg_parts, s2, g2, b2.reshape(1, c), npad, c)
  return out[:n]


# speedup vs baseline: 1.2315x; 1.2315x over previous
import jax
import jax.numpy as jnp
from jax import lax
from jax.experimental import pallas as pl
from jax.experimental.pallas import tpu as pltpu
from jax.experimental.pallas import tpu_sc as plsc

NC = 2
NS = 16
CHUNK = 128
CH_ALIGN = 160
MB = 256


def _zero_accum_slice(rows0, accum, base_r, n_row_blk, nz, d):
  def zr(i, _):
    rows0[i // (d // 16), pl.ds((i % (d // 16)) * 16, 16)] = jnp.zeros(
        (16,), jnp.float32)
    return 0
  lax.fori_loop(0, nz, zr, 0)

  def zb(k, _):
    pltpu.sync_copy(rows0, accum.at[pl.ds(base_r + k * CHUNK, CHUNK)])
    return 0
  lax.fori_loop(0, n_row_blk, zb, 0)


def _writeback(rows0, accum, out_ref, base_r, n_row_blk):
  def wb(k, _):
    r0 = base_r + k * CHUNK
    pltpu.sync_copy(accum.at[pl.ds(r0, CHUNK)], rows0)
    pltpu.sync_copy(rows0, out_ref.at[pl.ds(r0, CHUNK)])
    return 0
  lax.fori_loop(0, n_row_blk, wb, 0)


def _sc_edge_scatter_cols(table2, src2d, dst2d, npad, dh, ch_sub, nslot):
  rows_per_sub = npad // NS
  n_row_blk = rows_per_sub // CHUNK
  nz = CHUNK * (dh // 16)
  assert ch_sub % nslot == 0
  ngrp = ch_sub // nslot

  def body(table_hbm, src_hbm, dst_hbm, out_hbm, idx_s, idx_d, *rest):
    rows = list(rest[:nslot])
    accum = rest[nslot]
    gsem = list(rest[nslot + 1:2 * nslot + 1])
    cid = lax.axis_index("c")
    sid = lax.axis_index("s")
    base_r = sid * rows_per_sub
    base_c = sid * ch_sub

    _zero_accum_slice(rows[0], accum, base_r, n_row_blk, nz, dh)

    pltpu.sync_copy(src_hbm.at[pl.ds(base_c, ch_sub)], idx_s)
    pltpu.sync_copy(dst_hbm.at[pl.ds(base_c, ch_sub)], idx_d)
    plsc.subcore_barrier()

    def fire_g(j, b):
      pltpu.async_copy(table_hbm.at[cid].at[idx_s.at[j]], rows[b], gsem[b])

    def wait_g(j, b):
      pltpu.make_async_copy(table_hbm.at[cid].at[idx_s.at[j]], rows[b],
                            gsem[b]).wait()

    for b in range(nslot):
      fire_g(b, b)

    def grp(g, _):
      j0 = g * nslot
      for b in range(nslot):
        wait_g(j0 + b, b)
        pltpu.sync_copy(rows[b], accum.at[idx_d.at[j0 + b]], add=True)
        jn = jnp.minimum(j0 + nslot + b, ch_sub - 1)
        fire_g(jn, b)
      return 0
    lax.fori_loop(0, ngrp, grp, 0)
    for b in range(nslot):
      wait_g(ch_sub - 1, b)
    plsc.subcore_barrier()

    _writeback(rows[0], accum, out_hbm.at[cid], base_r, n_row_blk)

  return pl.kernel(
      body,
      out_type=jax.ShapeDtypeStruct((NC, npad, dh), jnp.float32),
      mesh=plsc.VectorSubcoreMesh(core_axis_name="c", subcore_axis_name="s"),
      compiler_params=pltpu.CompilerParams(use_tc_tiling_on_sc=False),
      scratch_types=(
          [pltpu.VMEM((ch_sub, CHUNK), jnp.int32),
           pltpu.VMEM((ch_sub, CHUNK), jnp.int32)]
          + [pltpu.VMEM((CHUNK, dh), jnp.float32) for _ in range(nslot)]
          + [pltpu.VMEM_SHARED((npad, dh), jnp.float32)]
          + [pltpu.SemaphoreType.DMA for _ in range(nslot)]
      ),
  )(table2, src2d, dst2d)


def _sc_edge_scatter_cols_res(table2, src2d, dst2d, npad, dh, ch_sub, nslot):
  rows_per_sub = npad // NS
  n_row_blk = rows_per_sub // CHUNK
  nz = CHUNK * (dh // 16)
  half = ch_sub // 2
  assert half % nslot == 0
  ngrp = half // nslot

  def body(table_hbm, src_hbm, dst_hbm, out_hbm, idx_s, idx_d, *rest):
    rows = list(rest[:nslot])
    tbl = rest[nslot]
    accum = rest[nslot + 1]
    gsem = list(rest[nslot + 2:2 * nslot + 2])
    cid = lax.axis_index("c")
    sid = lax.axis_index("s")
    base_r = sid * rows_per_sub
    base_c = sid * ch_sub

    _zero_accum_slice(rows[0], accum, base_r, n_row_blk, nz, dh)

    def ld(k, _):
      r0 = base_r + k * CHUNK
      pltpu.sync_copy(table_hbm.at[cid].at[pl.ds(r0, CHUNK)], rows[0])
      pltpu.sync_copy(rows[0], tbl.at[pl.ds(r0, CHUNK)])
      return 0
    lax.fori_loop(0, n_row_blk, ld, 0)

    pltpu.sync_copy(src_hbm.at[pl.ds(base_c, ch_sub)], idx_s)
    plsc.subcore_barrier()

    def fire_g(j, b):
      pltpu.async_copy(tbl.at[idx_s.at[j]], rows[b], gsem[b])

    def wait_g(j, b):
      pltpu.make_async_copy(tbl.at[idx_s.at[j]], rows[b], gsem[b]).wait()

    for phase in range(2):
      pltpu.sync_copy(dst_hbm.at[pl.ds(base_c + phase * half, half)], idx_d)
      p0 = phase * half
      for b in range(nslot):
        fire_g(p0 + b, b)

      def grp(g, _):
        j0 = g * nslot
        for b in range(nslot):
          wait_g(p0 + j0 + b, b)
          pltpu.sync_copy(rows[b], accum.at[idx_d.at[j0 + b]], add=True)
          jn = jnp.minimum(j0 + nslot + b, half - 1)
          fire_g(p0 + jn, b)
        return 0
      lax.fori_loop(0, ngrp, grp, 0)
      for b in range(nslot):
        wait_g(p0 + half - 1, b)
    plsc.subcore_barrier()

    _writeback(rows[0], accum, out_hbm.at[cid], base_r, n_row_blk)

  return pl.kernel(
      body,
      out_type=jax.ShapeDtypeStruct((NC, npad, dh), jnp.float32),
      mesh=plsc.VectorSubcoreMesh(core_axis_name="c", subcore_axis_name="s"),
      compiler_params=pltpu.CompilerParams(use_tc_tiling_on_sc=False),
      scratch_types=(
          [pltpu.VMEM((ch_sub, CHUNK), jnp.int32),
           pltpu.VMEM((ch_sub // 2, CHUNK), jnp.int32)]
          + [pltpu.VMEM((CHUNK, dh), jnp.float32) for _ in range(nslot)]
          + [pltpu.VMEM_SHARED((npad, dh), jnp.float32)]
          + [pltpu.VMEM_SHARED((npad, dh), jnp.float32)]
          + [pltpu.SemaphoreType.DMA for _ in range(nslot)]
      ),
  )(table2, src2d, dst2d)


def _sc_edge_scatter(table, src2d, dst2d, npad, d, ch_sub, nslot):
  rows_per_sub = npad // NS
  n_row_blk = rows_per_sub // CHUNK
  nz = CHUNK * (d // 16)
  chw = ch_sub // NC
  assert chw % nslot == 0
  ngrp = chw // nslot

  def body(table_hbm, src_hbm, dst_hbm, out_hbm, idx_s, idx_d, *rest):
    rows = list(rest[:nslot])
    accum = rest[nslot]
    gsem = list(rest[nslot + 1:2 * nslot + 1])
    cid = lax.axis_index("c")
    sid = lax.axis_index("s")
    base_r = sid * rows_per_sub
    base_c = sid * ch_sub + cid * chw

    _zero_accum_slice(rows[0], accum, base_r, n_row_blk, nz, d)

    pltpu.sync_copy(src_hbm.at[pl.ds(base_c, chw)], idx_s)
    pltpu.sync_copy(dst_hbm.at[pl.ds(base_c, chw)], idx_d)
    plsc.subcore_barrier()

    def fire_g(j, b):
      pltpu.async_copy(table_hbm.at[idx_s.at[j]], rows[b], gsem[b])

    def wait_g(j, b):
      pltpu.make_async_copy(table_hbm.at[idx_s.at[j]], rows[b],
                            gsem[b]).wait()

    for b in range(nslot):
      fire_g(b, b)

    def grp(g, _):
      j0 = g * nslot
      for b in range(nslot):
        wait_g(j0 + b, b)
        pltpu.sync_copy(rows[b], accum.at[idx_d.at[j0 + b]], add=True)
        jn = jnp.minimum(j0 + nslot + b, chw - 1)
        fire_g(jn, b)
      return 0
    lax.fori_loop(0, ngrp, grp, 0)
    for b in range(nslot):
      wait_g(chw - 1, b)
    plsc.subcore_barrier()

    _writeback(rows[0], accum, out_hbm.at[cid], base_r, n_row_blk)

  return pl.kernel(
      body,
      out_type=jax.ShapeDtypeStruct((NC, npad, d), jnp.float32),
      mesh=plsc.VectorSubcoreMesh(core_axis_name="c", subcore_axis_name="s"),
      compiler_params=pltpu.CompilerParams(use_tc_tiling_on_sc=False),
      scratch_types=(
          [pltpu.VMEM((chw, CHUNK), jnp.int32),
           pltpu.VMEM((chw, CHUNK), jnp.int32)]
          + [pltpu.VMEM((CHUNK, d), jnp.float32) for _ in range(nslot)]
          + [pltpu.VMEM_SHARED((npad, d), jnp.float32)]
          + [pltpu.SemaphoreType.DMA for _ in range(nslot)]
      ),
  )(table, src2d, dst2d)


def _sc_degree(dst2d, npad, ch_sub):
  d = 16
  rows_per_sub = npad // NS
  n_row_blk = rows_per_sub // CHUNK
  chw = ch_sub // NC

  def body(dst_hbm, out_hbm, idx_d, rows, accum):
    cid = lax.axis_index("c")
    sid = lax.axis_index("s")
    base_r = sid * rows_per_sub
    base_c = sid * ch_sub + cid * chw

    _zero_accum_slice(rows, accum, base_r, n_row_blk, CHUNK, d)

    def on(i, _):
      rows[i, pl.ds(0, 16)] = jnp.ones((16,), jnp.float32)
      return 0
    lax.fori_loop(0, CHUNK, on, 0)

    pltpu.sync_copy(dst_hbm.at[pl.ds(base_c, chw)], idx_d)
    plsc.subcore_barrier()

    def step(j, _):
      pltpu.sync_copy(rows, accum.at[idx_d.at[j]], add=True)
      return 0
    lax.fori_loop(0, chw, step, 0)
    plsc.subcore_barrier()

    _writeback(rows, accum, out_hbm.at[cid], base_r, n_row_blk)

  return pl.kernel(
      body,
      out_type=jax.ShapeDtypeStruct((NC, npad, d), jnp.float32),
      mesh=plsc.VectorSubcoreMesh(core_axis_name="c", subcore_axis_name="s"),
      compiler_params=pltpu.CompilerParams(use_tc_tiling_on_sc=False),
      scratch_types=[
          pltpu.VMEM((chw, CHUNK), jnp.int32),
          pltpu.VMEM((CHUNK, d), jnp.float32),
          pltpu.VMEM_SHARED((npad, d), jnp.float32),
      ],
  )(dst2d)


def _dinv_of(dp_ref):
  return lax.rsqrt(dp_ref[0, :, 0:1] + dp_ref[1, :, 0:1] + 1.0)


def _tc_layer1(deg_parts, x_pad, w1, npad, f, h):
  hh = h // 2

  def body(dp, xr, w1r, g1o):
    dinv = _dinv_of(dp)
    g1 = dinv * jnp.dot(xr[...], w1r[...], preferred_element_type=jnp.float32)
    g1o[0] = g1[:, :hh]
    g1o[1] = g1[:, hh:]
  return pl.pallas_call(
      body,
      grid=(npad // MB,),
      in_specs=[
          pl.BlockSpec((NC, MB, 16), lambda i: (0, i, 0)),
          pl.BlockSpec((MB, f), lambda i: (i, 0)),
          pl.BlockSpec((f, h), lambda i: (0, 0)),
      ],
      out_specs=pl.BlockSpec((NC, MB, hh), lambda i: (0, i, 0)),
      out_shape=jax.ShapeDtypeStruct((NC, npad, hh), jnp.float32),
  )(deg_parts, x_pad, w1)


def _tc_layer2(deg_parts, s1, g1, b1, w2, npad, h, c):
  hh = h // 2

  def body(dp, s1r, g1r, b1r, w2r, g2):
    dinv = _dinv_of(dp)
    m = jnp.concatenate([s1r[0] + g1r[0], s1r[1] + g1r[1]], axis=1)
    h1 = jnp.maximum(dinv * m + b1r[...], 0.0)
    g2[...] = dinv * jnp.dot(h1, w2r[...], preferred_element_type=jnp.float32)
  return pl.pallas_call(
      body,
      grid=(npad // MB,),
      in_specs=[
          pl.BlockSpec((NC, MB, 16), lambda i: (0, i, 0)),
          pl.BlockSpec((NC, MB, hh), lambda i: (0, i, 0)),
          pl.BlockSpec((NC, MB, hh), lambda i: (0, i, 0)),
          pl.BlockSpec((1, h), lambda i: (0, 0)),
          pl.BlockSpec((h, c), lambda i: (0, 0)),
      ],
      out_specs=pl.BlockSpec((MB, c), lambda i: (i, 0)),
      out_shape=jax.ShapeDtypeStruct((npad, c), jnp.float32),
  )(deg_parts, s1, g1, b1, w2)


def _tc_final(deg_parts, s2, g2, b2, npad, c):
  def body(dp, s2r, g2r, b2r, o):
    dinv = _dinv_of(dp)
    o[...] = dinv * (s2r[0] + s2r[1] + g2r[...]) + b2r[...]
  return pl.pallas_call(
      body,
      grid=(npad // MB,),
      in_specs=[
          pl.BlockSpec((NC, MB, 16), lambda i: (0, i, 0)),
          pl.BlockSpec((NC, MB, c), lambda i: (0, i, 0)),
          pl.BlockSpec((MB, c), lambda i: (i, 0)),
          pl.BlockSpec((1, c), lambda i: (0, 0)),
      ],
      out_specs=pl.BlockSpec((MB, c), lambda i: (i, 0)),
      out_shape=jax.ShapeDtypeStruct((npad, c), jnp.float32),
  )(deg_parts, s2, g2, b2)


def kernel(x, edge_index, W1, b1, W2, b2):
  n, f = x.shape
  h = W1.shape[1]
  c = W2.shape[1]
  e = edge_index.shape[1]

  npad = -(-(n + 1) // (NS * CHUNK)) * (NS * CHUNK)
  ch_min = -(-e // (NS * CHUNK))
  ch_sub = -(-ch_min // CH_ALIGN) * CH_ALIGN
  erows = ch_sub * NS
  epad = erows * CHUNK

  src = edge_index[0]
  dst = edge_index[1]
  pad_idx = jnp.full((epad - e,), n, dtype=jnp.int32)
  src2d = jnp.concatenate([src, pad_idx]).reshape(erows, CHUNK)
  dst2d = jnp.concatenate([dst, pad_idx]).reshape(erows, CHUNK)
  x_pad = jnp.pad(x, ((0, npad - n), (0, 0)))

  deg_parts = _sc_degree(dst2d, npad, ch_sub)
  g1 = _tc_layer1(deg_parts, x_pad, W1, npad, f, h)
  s1 = _sc_edge_scatter_cols_res(g1, src2d, dst2d, npad, h // 2, ch_sub, 2)
  g2 = _tc_layer2(deg_parts, s1, g1, b1.reshape(1, h), W2, npad, h, c)
  s2 = _sc_edge_scatter(g2, src2d, dst2d, npad, c, ch_sub, 16)
  out = _tc_final(deg_parts, s2, g2, b2.reshape(1, c), npad, c)
  return out[:n]


# speedup vs baseline: 1.3879x; 1.1270x over previous
import jax
import jax.numpy as jnp
from jax import lax
from jax.experimental import pallas as pl
from jax.experimental.pallas import tpu as pltpu
from jax.experimental.pallas import tpu_sc as plsc

NC = 2
NS = 16
CHUNK = 128
CH_ALIGN = 160
MB = 256


def _zero_accum_slice(rows0, accum, base_r, n_row_blk, nz, d):
  def zr(i, _):
    rows0[i // (d // 16), pl.ds((i % (d // 16)) * 16, 16)] = jnp.zeros(
        (16,), jnp.float32)
    return 0
  lax.fori_loop(0, nz, zr, 0)

  def zb(k, _):
    pltpu.sync_copy(rows0, accum.at[pl.ds(base_r + k * CHUNK, CHUNK)])
    return 0
  lax.fori_loop(0, n_row_blk, zb, 0)


def _writeback(rows0, accum, out_ref, base_r, n_row_blk):
  def wb(k, _):
    r0 = base_r + k * CHUNK
    pltpu.sync_copy(accum.at[pl.ds(r0, CHUNK)], rows0)
    pltpu.sync_copy(rows0, out_ref.at[pl.ds(r0, CHUNK)])
    return 0
  lax.fori_loop(0, n_row_blk, wb, 0)


def _sc_edge_scatter_cols(table2, src2d, dst2d, npad, dh, ch_sub, nslot):
  rows_per_sub = npad // NS
  n_row_blk = rows_per_sub // CHUNK
  nz = CHUNK * (dh // 16)
  assert ch_sub % nslot == 0
  ngrp = ch_sub // nslot

  def body(table_hbm, src_hbm, dst_hbm, out_hbm, idx_s, idx_d, *rest):
    rows = list(rest[:nslot])
    accum = rest[nslot]
    gsem = list(rest[nslot + 1:2 * nslot + 1])
    cid = lax.axis_index("c")
    sid = lax.axis_index("s")
    base_r = sid * rows_per_sub
    base_c = sid * ch_sub

    _zero_accum_slice(rows[0], accum, base_r, n_row_blk, nz, dh)

    pltpu.sync_copy(src_hbm.at[pl.ds(base_c, ch_sub)], idx_s)
    pltpu.sync_copy(dst_hbm.at[pl.ds(base_c, ch_sub)], idx_d)
    plsc.subcore_barrier()

    def fire_g(j, b):
      pltpu.async_copy(table_hbm.at[cid].at[idx_s.at[j]], rows[b], gsem[b])

    def wait_g(j, b):
      pltpu.make_async_copy(table_hbm.at[cid].at[idx_s.at[j]], rows[b],
                            gsem[b]).wait()

    for b in range(nslot):
      fire_g(b, b)

    def grp(g, _):
      j0 = g * nslot
      for b in range(nslot):
        wait_g(j0 + b, b)
        pltpu.sync_copy(rows[b], accum.at[idx_d.at[j0 + b]], add=True)
        jn = jnp.minimum(j0 + nslot + b, ch_sub - 1)
        fire_g(jn, b)
      return 0
    lax.fori_loop(0, ngrp, grp, 0)
    for b in range(nslot):
      wait_g(ch_sub - 1, b)
    plsc.subcore_barrier()

    _writeback(rows[0], accum, out_hbm.at[cid], base_r, n_row_blk)

  return pl.kernel(
      body,
      out_type=jax.ShapeDtypeStruct((NC, npad, dh), jnp.float32),
      mesh=plsc.VectorSubcoreMesh(core_axis_name="c", subcore_axis_name="s"),
      compiler_params=pltpu.CompilerParams(use_tc_tiling_on_sc=False),
      scratch_types=(
          [pltpu.VMEM((ch_sub, CHUNK), jnp.int32),
           pltpu.VMEM((ch_sub, CHUNK), jnp.int32)]
          + [pltpu.VMEM((CHUNK, dh), jnp.float32) for _ in range(nslot)]
          + [pltpu.VMEM_SHARED((npad, dh), jnp.float32)]
          + [pltpu.SemaphoreType.DMA for _ in range(nslot)]
      ),
  )(table2, src2d, dst2d)


def _sc_edge_scatter_cols_res(table2, src2d, dst2d, npad, dh, ch_sub, nslot):
  rows_per_sub = npad // NS
  n_row_blk = rows_per_sub // CHUNK
  nz = CHUNK * (dh // 16)
  half = ch_sub // 2
  assert half % nslot == 0
  ngrp = half // nslot

  def body(table_hbm, src_hbm, dst_hbm, out_hbm, idx_s, idx_d, *rest):
    rows = list(rest[:nslot])
    tbl = rest[nslot]
    accum = rest[nslot + 1]
    gsem = list(rest[nslot + 2:2 * nslot + 2])
    cid = lax.axis_index("c")
    sid = lax.axis_index("s")
    base_r = sid * rows_per_sub
    base_c = sid * ch_sub

    _zero_accum_slice(rows[0], accum, base_r, n_row_blk, nz, dh)

    def ld(k, _):
      r0 = base_r + k * CHUNK
      pltpu.sync_copy(table_hbm.at[cid].at[pl.ds(r0, CHUNK)], rows[0])
      pltpu.sync_copy(rows[0], tbl.at[pl.ds(r0, CHUNK)])
      return 0
    lax.fori_loop(0, n_row_blk, ld, 0)

    pltpu.sync_copy(src_hbm.at[pl.ds(base_c, ch_sub)], idx_s)
    plsc.subcore_barrier()

    def fire_g(j, b):
      pltpu.async_copy(tbl.at[idx_s.at[j]], rows[b], gsem[b])

    def wait_g(j, b):
      pltpu.make_async_copy(tbl.at[idx_s.at[j]], rows[b], gsem[b]).wait()

    for phase in range(2):
      pltpu.sync_copy(dst_hbm.at[pl.ds(base_c + phase * half, half)], idx_d)
      p0 = phase * half
      for b in range(nslot):
        fire_g(p0 + b, b)

      def grp(g, _):
        j0 = g * nslot
        for b in range(nslot):
          wait_g(p0 + j0 + b, b)
          pltpu.sync_copy(rows[b], accum.at[idx_d.at[j0 + b]], add=True)
          jn = jnp.minimum(j0 + nslot + b, half - 1)
          fire_g(p0 + jn, b)
        return 0
      lax.fori_loop(0, ngrp, grp, 0)
      for b in range(nslot):
        wait_g(p0 + half - 1, b)
    plsc.subcore_barrier()

    _writeback(rows[0], accum, out_hbm.at[cid], base_r, n_row_blk)

  return pl.kernel(
      body,
      out_type=jax.ShapeDtypeStruct((NC, npad, dh), jnp.float32),
      mesh=plsc.VectorSubcoreMesh(core_axis_name="c", subcore_axis_name="s"),
      compiler_params=pltpu.CompilerParams(use_tc_tiling_on_sc=False),
      scratch_types=(
          [pltpu.VMEM((ch_sub, CHUNK), jnp.int32),
           pltpu.VMEM((ch_sub // 2, CHUNK), jnp.int32)]
          + [pltpu.VMEM((CHUNK, dh), jnp.float32) for _ in range(nslot)]
          + [pltpu.VMEM_SHARED((npad, dh), jnp.float32)]
          + [pltpu.VMEM_SHARED((npad, dh), jnp.float32)]
          + [pltpu.SemaphoreType.DMA for _ in range(nslot)]
      ),
  )(table2, src2d, dst2d)


def _sc_edge_scatter_res(table, src2d, dst2d, npad, d, ch_sub, nslot):
  rows_per_sub = npad // NS
  n_row_blk = rows_per_sub // CHUNK
  nz = CHUNK * (d // 16)
  chw = ch_sub // NC
  assert chw % nslot == 0
  ngrp = chw // nslot

  def body(table_hbm, src_hbm, dst_hbm, out_hbm, idx_s, idx_d, *rest):
    rows = list(rest[:nslot])
    tbl = rest[nslot]
    accum = rest[nslot + 1]
    gsem = list(rest[nslot + 2:2 * nslot + 2])
    cid = lax.axis_index("c")
    sid = lax.axis_index("s")
    base_r = sid * rows_per_sub
    base_c = sid * ch_sub + cid * chw

    _zero_accum_slice(rows[0], accum, base_r, n_row_blk, nz, d)

    def ld(k, _):
      r0 = base_r + k * CHUNK
      pltpu.sync_copy(table_hbm.at[pl.ds(r0, CHUNK)], rows[0])
      pltpu.sync_copy(rows[0], tbl.at[pl.ds(r0, CHUNK)])
      return 0
    lax.fori_loop(0, n_row_blk, ld, 0)

    pltpu.sync_copy(src_hbm.at[pl.ds(base_c, chw)], idx_s)
    pltpu.sync_copy(dst_hbm.at[pl.ds(base_c, chw)], idx_d)
    plsc.subcore_barrier()

    def fire_g(j, b):
      pltpu.async_copy(tbl.at[idx_s.at[j]], rows[b], gsem[b])

    def wait_g(j, b):
      pltpu.make_async_copy(tbl.at[idx_s.at[j]], rows[b], gsem[b]).wait()

    for b in range(nslot):
      fire_g(b, b)

    def grp(g, _):
      j0 = g * nslot
      for b in range(nslot):
        wait_g(j0 + b, b)
        pltpu.sync_copy(rows[b], accum.at[idx_d.at[j0 + b]], add=True)
        jn = jnp.minimum(j0 + nslot + b, chw - 1)
        fire_g(jn, b)
      return 0
    lax.fori_loop(0, ngrp, grp, 0)
    for b in range(nslot):
      wait_g(chw - 1, b)
    plsc.subcore_barrier()

    _writeback(rows[0], accum, out_hbm.at[cid], base_r, n_row_blk)

  return pl.kernel(
      body,
      out_type=jax.ShapeDtypeStruct((NC, npad, d), jnp.float32),
      mesh=plsc.VectorSubcoreMesh(core_axis_name="c", subcore_axis_name="s"),
      compiler_params=pltpu.CompilerParams(use_tc_tiling_on_sc=False),
      scratch_types=(
          [pltpu.VMEM((chw, CHUNK), jnp.int32),
           pltpu.VMEM((chw, CHUNK), jnp.int32)]
          + [pltpu.VMEM((CHUNK, d), jnp.float32) for _ in range(nslot)]
          + [pltpu.VMEM_SHARED((npad, d), jnp.float32)]
          + [pltpu.VMEM_SHARED((npad, d), jnp.float32)]
          + [pltpu.SemaphoreType.DMA for _ in range(nslot)]
      ),
  )(table, src2d, dst2d)


def _sc_edge_scatter(table, src2d, dst2d, npad, d, ch_sub, nslot):
  rows_per_sub = npad // NS
  n_row_blk = rows_per_sub // CHUNK
  nz = CHUNK * (d // 16)
  chw = ch_sub // NC
  assert chw % nslot == 0
  ngrp = chw // nslot

  def body(table_hbm, src_hbm, dst_hbm, out_hbm, idx_s, idx_d, *rest):
    rows = list(rest[:nslot])
    accum = rest[nslot]
    gsem = list(rest[nslot + 1:2 * nslot + 1])
    cid = lax.axis_index("c")
    sid = lax.axis_index("s")
    base_r = sid * rows_per_sub
    base_c = sid * ch_sub + cid * chw

    _zero_accum_slice(rows[0], accum, base_r, n_row_blk, nz, d)

    pltpu.sync_copy(src_hbm.at[pl.ds(base_c, chw)], idx_s)
    pltpu.sync_copy(dst_hbm.at[pl.ds(base_c, chw)], idx_d)
    plsc.subcore_barrier()

    def fire_g(j, b):
      pltpu.async_copy(table_hbm.at[idx_s.at[j]], rows[b], gsem[b])

    def wait_g(j, b):
      pltpu.make_async_copy(table_hbm.at[idx_s.at[j]], rows[b],
                            gsem[b]).wait()

    for b in range(nslot):
      fire_g(b, b)

    def grp(g, _):
      j0 = g * nslot
      for b in range(nslot):
        wait_g(j0 + b, b)
        pltpu.sync_copy(rows[b], accum.at[idx_d.at[j0 + b]], add=True)
        jn = jnp.minimum(j0 + nslot + b, chw - 1)
        fire_g(jn, b)
      return 0
    lax.fori_loop(0, ngrp, grp, 0)
    for b in range(nslot):
      wait_g(chw - 1, b)
    plsc.subcore_barrier()

    _writeback(rows[0], accum, out_hbm.at[cid], base_r, n_row_blk)

  return pl.kernel(
      body,
      out_type=jax.ShapeDtypeStruct((NC, npad, d), jnp.float32),
      mesh=plsc.VectorSubcoreMesh(core_axis_name="c", subcore_axis_name="s"),
      compiler_params=pltpu.CompilerParams(use_tc_tiling_on_sc=False),
      scratch_types=(
          [pltpu.VMEM((chw, CHUNK), jnp.int32),
           pltpu.VMEM((chw, CHUNK), jnp.int32)]
          + [pltpu.VMEM((CHUNK, d), jnp.float32) for _ in range(nslot)]
          + [pltpu.VMEM_SHARED((npad, d), jnp.float32)]
          + [pltpu.SemaphoreType.DMA for _ in range(nslot)]
      ),
  )(table, src2d, dst2d)


def _sc_degree(dst2d, npad, ch_sub):
  d = 16
  rows_per_sub = npad // NS
  n_row_blk = rows_per_sub // CHUNK
  chw = ch_sub // NC

  def body(dst_hbm, out_hbm, idx_d, rows, accum):
    cid = lax.axis_index("c")
    sid = lax.axis_index("s")
    base_r = sid * rows_per_sub
    base_c = sid * ch_sub + cid * chw

    _zero_accum_slice(rows, accum, base_r, n_row_blk, CHUNK, d)

    def on(i, _):
      rows[i, pl.ds(0, 16)] = jnp.ones((16,), jnp.float32)
      return 0
    lax.fori_loop(0, CHUNK, on, 0)

    pltpu.sync_copy(dst_hbm.at[pl.ds(base_c, chw)], idx_d)
    plsc.subcore_barrier()

    def step(j, _):
      pltpu.sync_copy(rows, accum.at[idx_d.at[j]], add=True)
      return 0
    lax.fori_loop(0, chw, step, 0)
    plsc.subcore_barrier()

    _writeback(rows, accum, out_hbm.at[cid], base_r, n_row_blk)

  return pl.kernel(
      body,
      out_type=jax.ShapeDtypeStruct((NC, npad, d), jnp.float32),
      mesh=plsc.VectorSubcoreMesh(core_axis_name="c", subcore_axis_name="s"),
      compiler_params=pltpu.CompilerParams(use_tc_tiling_on_sc=False),
      scratch_types=[
          pltpu.VMEM((chw, CHUNK), jnp.int32),
          pltpu.VMEM((CHUNK, d), jnp.float32),
          pltpu.VMEM_SHARED((npad, d), jnp.float32),
      ],
  )(dst2d)


def _dinv_of(dp_ref):
  return lax.rsqrt(dp_ref[0, :, 0:1] + dp_ref[1, :, 0:1] + 1.0)


def _tc_layer1(deg_parts, x_pad, w1, npad, f, h):
  hh = h // 2

  def body(dp, xr, w1r, g1o):
    dinv = _dinv_of(dp)
    g1 = dinv * jnp.dot(xr[...], w1r[...], preferred_element_type=jnp.float32)
    g1o[0] = g1[:, :hh]
    g1o[1] = g1[:, hh:]
  return pl.pallas_call(
      body,
      grid=(npad // MB,),
      in_specs=[
          pl.BlockSpec((NC, MB, 16), lambda i: (0, i, 0)),
          pl.BlockSpec((MB, f), lambda i: (i, 0)),
          pl.BlockSpec((f, h), lambda i: (0, 0)),
      ],
      out_specs=pl.BlockSpec((NC, MB, hh), lambda i: (0, i, 0)),
      out_shape=jax.ShapeDtypeStruct((NC, npad, hh), jnp.float32),
  )(deg_parts, x_pad, w1)


def _tc_layer2(deg_parts, s1, g1, b1, w2, npad, h, c):
  hh = h // 2

  def body(dp, s1r, g1r, b1r, w2r, g2):
    dinv = _dinv_of(dp)
    m = jnp.concatenate([s1r[0] + g1r[0], s1r[1] + g1r[1]], axis=1)
    h1 = jnp.maximum(dinv * m + b1r[...], 0.0)
    g2[...] = dinv * jnp.dot(h1, w2r[...], preferred_element_type=jnp.float32)
  return pl.pallas_call(
      body,
      grid=(npad // MB,),
      in_specs=[
          pl.BlockSpec((NC, MB, 16), lambda i: (0, i, 0)),
          pl.BlockSpec((NC, MB, hh), lambda i: (0, i, 0)),
          pl.BlockSpec((NC, MB, hh), lambda i: (0, i, 0)),
          pl.BlockSpec((1, h), lambda i: (0, 0)),
          pl.BlockSpec((h, c), lambda i: (0, 0)),
      ],
      out_specs=pl.BlockSpec((MB, c), lambda i: (i, 0)),
      out_shape=jax.ShapeDtypeStruct((npad, c), jnp.float32),
  )(deg_parts, s1, g1, b1, w2)


def _tc_final(deg_parts, s2, g2, b2, npad, c):
  def body(dp, s2r, g2r, b2r, o):
    dinv = _dinv_of(dp)
    o[...] = dinv * (s2r[0] + s2r[1] + g2r[...]) + b2r[...]
  return pl.pallas_call(
      body,
      grid=(npad // MB,),
      in_specs=[
          pl.BlockSpec((NC, MB, 16), lambda i: (0, i, 0)),
          pl.BlockSpec((NC, MB, c), lambda i: (0, i, 0)),
          pl.BlockSpec((MB, c), lambda i: (i, 0)),
          pl.BlockSpec((1, c), lambda i: (0, 0)),
      ],
      out_specs=pl.BlockSpec((MB, c), lambda i: (i, 0)),
      out_shape=jax.ShapeDtypeStruct((npad, c), jnp.float32),
  )(deg_parts, s2, g2, b2)


def kernel(x, edge_index, W1, b1, W2, b2):
  n, f = x.shape
  h = W1.shape[1]
  c = W2.shape[1]
  e = edge_index.shape[1]

  npad = -(-(n + 1) // (NS * CHUNK)) * (NS * CHUNK)
  ch_min = -(-e // (NS * CHUNK))
  ch_sub = -(-ch_min // CH_ALIGN) * CH_ALIGN
  erows = ch_sub * NS
  epad = erows * CHUNK

  src = edge_index[0]
  dst = edge_index[1]
  pad_idx = jnp.full((epad - e,), n, dtype=jnp.int32)
  src2d = jnp.concatenate([src, pad_idx]).reshape(erows, CHUNK)
  dst2d = jnp.concatenate([dst, pad_idx]).reshape(erows, CHUNK)
  x_pad = jnp.pad(x, ((0, npad - n), (0, 0)))

  deg_parts = _sc_degree(dst2d, npad, ch_sub)
  g1 = _tc_layer1(deg_parts, x_pad, W1, npad, f, h)
  s1 = _sc_edge_scatter_cols_res(g1, src2d, dst2d, npad, h // 2, ch_sub, 2)
  g2 = _tc_layer2(deg_parts, s1, g1, b1.reshape(1, h), W2, npad, h, c)
  s2 = _sc_edge_scatter_res(g2, src2d, dst2d, npad, c, ch_sub, 16)
  out = _tc_final(deg_parts, s2, g2, b2.reshape(1, c), npad, c)
  return out[:n]


# speedup vs baseline: 1.4165x; 1.0206x over previous
import jax
import jax.numpy as jnp
from jax import lax
from jax.experimental import pallas as pl
from jax.experimental.pallas import tpu as pltpu
from jax.experimental.pallas import tpu_sc as plsc

NC = 2
NS = 16
CHUNK = 128
CH_ALIGN = 160
MB = 256


def _zero_accum_slice(rows0, accum, base_r, n_row_blk, nz, d):
  def zr(i, _):
    rows0[i // (d // 16), pl.ds((i % (d // 16)) * 16, 16)] = jnp.zeros(
        (16,), jnp.float32)
    return 0
  lax.fori_loop(0, nz, zr, 0)

  def zb(k, _):
    pltpu.sync_copy(rows0, accum.at[pl.ds(base_r + k * CHUNK, CHUNK)])
    return 0
  lax.fori_loop(0, n_row_blk, zb, 0)


def _writeback(rows0, accum, out_ref, base_r, n_row_blk):
  del rows0
  pltpu.sync_copy(accum.at[pl.ds(base_r, n_row_blk * CHUNK)],
                  out_ref.at[pl.ds(base_r, n_row_blk * CHUNK)])


def _sc_edge_scatter_cols(table2, src2d, dst2d, npad, dh, ch_sub, nslot):
  rows_per_sub = npad // NS
  n_row_blk = rows_per_sub // CHUNK
  nz = CHUNK * (dh // 16)
  assert ch_sub % nslot == 0
  ngrp = ch_sub // nslot

  def body(table_hbm, src_hbm, dst_hbm, out_hbm, idx_s, idx_d, *rest):
    rows = list(rest[:nslot])
    accum = rest[nslot]
    gsem = list(rest[nslot + 1:2 * nslot + 1])
    cid = lax.axis_index("c")
    sid = lax.axis_index("s")
    base_r = sid * rows_per_sub
    base_c = sid * ch_sub

    _zero_accum_slice(rows[0], accum, base_r, n_row_blk, nz, dh)

    pltpu.sync_copy(src_hbm.at[pl.ds(base_c, ch_sub)], idx_s)
    pltpu.sync_copy(dst_hbm.at[pl.ds(base_c, ch_sub)], idx_d)
    plsc.subcore_barrier()

    def fire_g(j, b):
      pltpu.async_copy(table_hbm.at[cid].at[idx_s.at[j]], rows[b], gsem[b])

    def wait_g(j, b):
      pltpu.make_async_copy(table_hbm.at[cid].at[idx_s.at[j]], rows[b],
                            gsem[b]).wait()

    for b in range(nslot):
      fire_g(b, b)

    def grp(g, _):
      j0 = g * nslot
      for b in range(nslot):
        wait_g(j0 + b, b)
        pltpu.sync_copy(rows[b], accum.at[idx_d.at[j0 + b]], add=True)
        jn = jnp.minimum(j0 + nslot + b, ch_sub - 1)
        fire_g(jn, b)
      return 0
    lax.fori_loop(0, ngrp, grp, 0)
    for b in range(nslot):
      wait_g(ch_sub - 1, b)
    plsc.subcore_barrier()

    _writeback(rows[0], accum, out_hbm.at[cid], base_r, n_row_blk)

  return pl.kernel(
      body,
      out_type=jax.ShapeDtypeStruct((NC, npad, dh), jnp.float32),
      mesh=plsc.VectorSubcoreMesh(core_axis_name="c", subcore_axis_name="s"),
      compiler_params=pltpu.CompilerParams(use_tc_tiling_on_sc=False),
      scratch_types=(
          [pltpu.VMEM((ch_sub, CHUNK), jnp.int32),
           pltpu.VMEM((ch_sub, CHUNK), jnp.int32)]
          + [pltpu.VMEM((CHUNK, dh), jnp.float32) for _ in range(nslot)]
          + [pltpu.VMEM_SHARED((npad, dh), jnp.float32)]
          + [pltpu.SemaphoreType.DMA for _ in range(nslot)]
      ),
  )(table2, src2d, dst2d)


def _sc_edge_scatter_cols_res(table2, src2d, dst2d, npad, dh, ch_sub, nslot):
  rows_per_sub = npad // NS
  n_row_blk = rows_per_sub // CHUNK
  nz = CHUNK * (dh // 16)
  half = ch_sub // 2
  assert half % nslot == 0
  ngrp = half // nslot

  def body(table_hbm, src_hbm, dst_hbm, out_hbm, idx_s, idx_d, *rest):
    rows = list(rest[:nslot])
    tbl = rest[nslot]
    accum = rest[nslot + 1]
    gsem = list(rest[nslot + 2:2 * nslot + 2])
    cid = lax.axis_index("c")
    sid = lax.axis_index("s")
    base_r = sid * rows_per_sub
    base_c = sid * ch_sub

    _zero_accum_slice(rows[0], accum, base_r, n_row_blk, nz, dh)

    pltpu.sync_copy(table_hbm.at[cid].at[pl.ds(base_r, rows_per_sub)],
                    tbl.at[pl.ds(base_r, rows_per_sub)])

    pltpu.sync_copy(src_hbm.at[pl.ds(base_c, ch_sub)], idx_s)
    plsc.subcore_barrier()

    def fire_g(j, b):
      pltpu.async_copy(tbl.at[idx_s.at[j]], rows[b], gsem[b])

    def wait_g(j, b):
      pltpu.make_async_copy(tbl.at[idx_s.at[j]], rows[b], gsem[b]).wait()

    for phase in range(2):
      pltpu.sync_copy(dst_hbm.at[pl.ds(base_c + phase * half, half)], idx_d)
      p0 = phase * half
      for b in range(nslot):
        fire_g(p0 + b, b)

      def grp(g, _):
        j0 = g * nslot
        for b in range(nslot):
          wait_g(p0 + j0 + b, b)
          pltpu.sync_copy(rows[b], accum.at[idx_d.at[j0 + b]], add=True)
          jn = jnp.minimum(j0 + nslot + b, half - 1)
          fire_g(p0 + jn, b)
        return 0
      lax.fori_loop(0, ngrp, grp, 0)
      for b in range(nslot):
        wait_g(p0 + half - 1, b)
    plsc.subcore_barrier()

    _writeback(rows[0], accum, out_hbm.at[cid], base_r, n_row_blk)

  return pl.kernel(
      body,
      out_type=jax.ShapeDtypeStruct((NC, npad, dh), jnp.float32),
      mesh=plsc.VectorSubcoreMesh(core_axis_name="c", subcore_axis_name="s"),
      compiler_params=pltpu.CompilerParams(use_tc_tiling_on_sc=False),
      scratch_types=(
          [pltpu.VMEM((ch_sub, CHUNK), jnp.int32),
           pltpu.VMEM((ch_sub // 2, CHUNK), jnp.int32)]
          + [pltpu.VMEM((CHUNK, dh), jnp.float32) for _ in range(nslot)]
          + [pltpu.VMEM_SHARED((npad, dh), jnp.float32)]
          + [pltpu.VMEM_SHARED((npad, dh), jnp.float32)]
          + [pltpu.SemaphoreType.DMA for _ in range(nslot)]
      ),
  )(table2, src2d, dst2d)


def _sc_edge_scatter_res(table, src2d, dst2d, npad, d, ch_sub, nslot):
  rows_per_sub = npad // NS
  n_row_blk = rows_per_sub // CHUNK
  nz = CHUNK * (d // 16)
  chw = ch_sub // NC
  assert chw % nslot == 0
  ngrp = chw // nslot

  def body(table_hbm, src_hbm, dst_hbm, out_hbm, idx_s, idx_d, *rest):
    rows = list(rest[:nslot])
    tbl = rest[nslot]
    accum = rest[nslot + 1]
    gsem = list(rest[nslot + 2:2 * nslot + 2])
    cid = lax.axis_index("c")
    sid = lax.axis_index("s")
    base_r = sid * rows_per_sub
    base_c = sid * ch_sub + cid * chw

    _zero_accum_slice(rows[0], accum, base_r, n_row_blk, nz, d)

    pltpu.sync_copy(table_hbm.at[pl.ds(base_r, rows_per_sub)],
                    tbl.at[pl.ds(base_r, rows_per_sub)])

    pltpu.sync_copy(src_hbm.at[pl.ds(base_c, chw)], idx_s)
    pltpu.sync_copy(dst_hbm.at[pl.ds(base_c, chw)], idx_d)
    plsc.subcore_barrier()

    def fire_g(j, b):
      pltpu.async_copy(tbl.at[idx_s.at[j]], rows[b], gsem[b])

    def wait_g(j, b):
      pltpu.make_async_copy(tbl.at[idx_s.at[j]], rows[b], gsem[b]).wait()

    for b in range(nslot):
      fire_g(b, b)

    def grp(g, _):
      j0 = g * nslot
      for b in range(nslot):
        wait_g(j0 + b, b)
        pltpu.sync_copy(rows[b], accum.at[idx_d.at[j0 + b]], add=True)
        jn = jnp.minimum(j0 + nslot + b, chw - 1)
        fire_g(jn, b)
      return 0
    lax.fori_loop(0, ngrp, grp, 0)
    for b in range(nslot):
      wait_g(chw - 1, b)
    plsc.subcore_barrier()

    _writeback(rows[0], accum, out_hbm.at[cid], base_r, n_row_blk)

  return pl.kernel(
      body,
      out_type=jax.ShapeDtypeStruct((NC, npad, d), jnp.float32),
      mesh=plsc.VectorSubcoreMesh(core_axis_name="c", subcore_axis_name="s"),
      compiler_params=pltpu.CompilerParams(use_tc_tiling_on_sc=False),
      scratch_types=(
          [pltpu.VMEM((chw, CHUNK), jnp.int32),
           pltpu.VMEM((chw, CHUNK), jnp.int32)]
          + [pltpu.VMEM((CHUNK, d), jnp.float32) for _ in range(nslot)]
          + [pltpu.VMEM_SHARED((npad, d), jnp.float32)]
          + [pltpu.VMEM_SHARED((npad, d), jnp.float32)]
          + [pltpu.SemaphoreType.DMA for _ in range(nslot)]
      ),
  )(table, src2d, dst2d)


def _sc_edge_scatter(table, src2d, dst2d, npad, d, ch_sub, nslot):
  rows_per_sub = npad // NS
  n_row_blk = rows_per_sub // CHUNK
  nz = CHUNK * (d // 16)
  chw = ch_sub // NC
  assert chw % nslot == 0
  ngrp = chw // nslot

  def body(table_hbm, src_hbm, dst_hbm, out_hbm, idx_s, idx_d, *rest):
    rows = list(rest[:nslot])
    accum = rest[nslot]
    gsem = list(rest[nslot + 1:2 * nslot + 1])
    cid = lax.axis_index("c")
    sid = lax.axis_index("s")
    base_r = sid * rows_per_sub
    base_c = sid * ch_sub + cid * chw

    _zero_accum_slice(rows[0], accum, base_r, n_row_blk, nz, d)

    pltpu.sync_copy(src_hbm.at[pl.ds(base_c, chw)], idx_s)
    pltpu.sync_copy(dst_hbm.at[pl.ds(base_c, chw)], idx_d)
    plsc.subcore_barrier()

    def fire_g(j, b):
      pltpu.async_copy(table_hbm.at[idx_s.at[j]], rows[b], gsem[b])

    def wait_g(j, b):
      pltpu.make_async_copy(table_hbm.at[idx_s.at[j]], rows[b],
                            gsem[b]).wait()

    for b in range(nslot):
      fire_g(b, b)

    def grp(g, _):
      j0 = g * nslot
      for b in range(nslot):
        wait_g(j0 + b, b)
        pltpu.sync_copy(rows[b], accum.at[idx_d.at[j0 + b]], add=True)
        jn = jnp.minimum(j0 + nslot + b, chw - 1)
        fire_g(jn, b)
      return 0
    lax.fori_loop(0, ngrp, grp, 0)
    for b in range(nslot):
      wait_g(chw - 1, b)
    plsc.subcore_barrier()

    _writeback(rows[0], accum, out_hbm.at[cid], base_r, n_row_blk)

  return pl.kernel(
      body,
      out_type=jax.ShapeDtypeStruct((NC, npad, d), jnp.float32),
      mesh=plsc.VectorSubcoreMesh(core_axis_name="c", subcore_axis_name="s"),
      compiler_params=pltpu.CompilerParams(use_tc_tiling_on_sc=False),
      scratch_types=(
          [pltpu.VMEM((chw, CHUNK), jnp.int32),
           pltpu.VMEM((chw, CHUNK), jnp.int32)]
          + [pltpu.VMEM((CHUNK, d), jnp.float32) for _ in range(nslot)]
          + [pltpu.VMEM_SHARED((npad, d), jnp.float32)]
          + [pltpu.SemaphoreType.DMA for _ in range(nslot)]
      ),
  )(table, src2d, dst2d)


def _sc_degree(dst2d, npad, ch_sub):
  d = 16
  rows_per_sub = npad // NS
  n_row_blk = rows_per_sub // CHUNK
  chw = ch_sub // NC

  def body(dst_hbm, out_hbm, idx_d, rows, accum):
    cid = lax.axis_index("c")
    sid = lax.axis_index("s")
    base_r = sid * rows_per_sub
    base_c = sid * ch_sub + cid * chw

    _zero_accum_slice(rows, accum, base_r, n_row_blk, CHUNK, d)

    def on(i, _):
      rows[i, pl.ds(0, 16)] = jnp.ones((16,), jnp.float32)
      return 0
    lax.fori_loop(0, CHUNK, on, 0)

    pltpu.sync_copy(dst_hbm.at[pl.ds(base_c, chw)], idx_d)
    plsc.subcore_barrier()

    def step(j, _):
      pltpu.sync_copy(rows, accum.at[idx_d.at[j]], add=True)
      return 0
    lax.fori_loop(0, chw, step, 0)
    plsc.subcore_barrier()

    _writeback(rows, accum, out_hbm.at[cid], base_r, n_row_blk)

  return pl.kernel(
      body,
      out_type=jax.ShapeDtypeStruct((NC, npad, d), jnp.float32),
      mesh=plsc.VectorSubcoreMesh(core_axis_name="c", subcore_axis_name="s"),
      compiler_params=pltpu.CompilerParams(use_tc_tiling_on_sc=False),
      scratch_types=[
          pltpu.VMEM((chw, CHUNK), jnp.int32),
          pltpu.VMEM((CHUNK, d), jnp.float32),
          pltpu.VMEM_SHARED((npad, d), jnp.float32),
      ],
  )(dst2d)


def _dinv_of(dp_ref):
  return lax.rsqrt(dp_ref[0, :, 0:1] + dp_ref[1, :, 0:1] + 1.0)


def _tc_layer1(deg_parts, x_pad, w1, npad, f, h):
  hh = h // 2

  def body(dp, xr, w1r, g1o):
    dinv = _dinv_of(dp)
    g1 = dinv * jnp.dot(xr[...], w1r[...], preferred_element_type=jnp.float32)
    g1o[0] = g1[:, :hh]
    g1o[1] = g1[:, hh:]
  return pl.pallas_call(
      body,
      grid=(npad // MB,),
      in_specs=[
          pl.BlockSpec((NC, MB, 16), lambda i: (0, i, 0)),
          pl.BlockSpec((MB, f), lambda i: (i, 0)),
          pl.BlockSpec((f, h), lambda i: (0, 0)),
      ],
      out_specs=pl.BlockSpec((NC, MB, hh), lambda i: (0, i, 0)),
      out_shape=jax.ShapeDtypeStruct((NC, npad, hh), jnp.float32),
  )(deg_parts, x_pad, w1)


def _tc_layer2(deg_parts, s1, g1, b1, w2, npad, h, c):
  hh = h // 2

  def body(dp, s1r, g1r, b1r, w2r, g2):
    dinv = _dinv_of(dp)
    m = jnp.concatenate([s1r[0] + g1r[0], s1r[1] + g1r[1]], axis=1)
    h1 = jnp.maximum(dinv * m + b1r[...], 0.0)
    g2[...] = dinv * jnp.dot(h1, w2r[...], preferred_element_type=jnp.float32)
  return pl.pallas_call(
      body,
      grid=(npad // MB,),
      in_specs=[
          pl.BlockSpec((NC, MB, 16), lambda i: (0, i, 0)),
          pl.BlockSpec((NC, MB, hh), lambda i: (0, i, 0)),
          pl.BlockSpec((NC, MB, hh), lambda i: (0, i, 0)),
          pl.BlockSpec((1, h), lambda i: (0, 0)),
          pl.BlockSpec((h, c), lambda i: (0, 0)),
      ],
      out_specs=pl.BlockSpec((MB, c), lambda i: (i, 0)),
      out_shape=jax.ShapeDtypeStruct((npad, c), jnp.float32),
  )(deg_parts, s1, g1, b1, w2)


def _tc_final(deg_parts, s2, g2, b2, npad, c):
  def body(dp, s2r, g2r, b2r, o):
    dinv = _dinv_of(dp)
    o[...] = dinv * (s2r[0] + s2r[1] + g2r[...]) + b2r[...]
  return pl.pallas_call(
      body,
      grid=(npad // MB,),
      in_specs=[
          pl.BlockSpec((NC, MB, 16), lambda i: (0, i, 0)),
          pl.BlockSpec((NC, MB, c), lambda i: (0, i, 0)),
          pl.BlockSpec((MB, c), lambda i: (i, 0)),
          pl.BlockSpec((1, c), lambda i: (0, 0)),
      ],
      out_specs=pl.BlockSpec((MB, c), lambda i: (i, 0)),
      out_shape=jax.ShapeDtypeStruct((npad, c), jnp.float32),
  )(deg_parts, s2, g2, b2)


def kernel(x, edge_index, W1, b1, W2, b2):
  n, f = x.shape
  h = W1.shape[1]
  c = W2.shape[1]
  e = edge_index.shape[1]

  npad = -(-(n + 1) // (NS * CHUNK)) * (NS * CHUNK)
  ch_min = -(-e // (NS * CHUNK))
  ch_sub = -(-ch_min // CH_ALIGN) * CH_ALIGN
  erows = ch_sub * NS
  epad = erows * CHUNK

  src = edge_index[0]
  dst = edge_index[1]
  pad_idx = jnp.full((epad - e,), n, dtype=jnp.int32)
  src2d = jnp.concatenate([src, pad_idx]).reshape(erows, CHUNK)
  dst2d = jnp.concatenate([dst, pad_idx]).reshape(erows, CHUNK)
  x_pad = jnp.pad(x, ((0, npad - n), (0, 0)))

  deg_parts = _sc_degree(dst2d, npad, ch_sub)
  g1 = _tc_layer1(deg_parts, x_pad, W1, npad, f, h)
  s1 = _sc_edge_scatter_cols_res(g1, src2d, dst2d, npad, h // 2, ch_sub, 2)
  g2 = _tc_layer2(deg_parts, s1, g1, b1.reshape(1, h), W2, npad, h, c)
  s2 = _sc_edge_scatter_res(g2, src2d, dst2d, npad, c, ch_sub, 16)
  out = _tc_final(deg_parts, s2, g2, b2.reshape(1, c), npad, c)
  return out[:n]
